# baseline (speedup 1.0000x reference)
import jax
import jax.numpy as jnp
from jax import lax
from jax.experimental import pallas as pl
from jax.experimental.pallas import tpu as pltpu


ROWS = 4096
COLS = 20
VOCAB = 1000
NBUF = 2
K = 5
CH = VOCAB // K


def _dma(scratch, out_hbm, sems, h, k, c):
    return pltpu.make_async_copy(
        scratch.at[h, pl.ds(k * CH, CH)],
        out_hbm.at[c, pl.ds(k * CH, CH)],
        sems.at[h, k],
    )


def _body(x_ref, out_hbm, scratch, sems):
    c = pl.program_id(0)
    h = lax.rem(c, 2)

    @pl.when(c >= NBUF)
    def _wait_prev():
        for k in range(K):
            _dma(scratch, out_hbm, sems, h, k, c - NBUF).wait()

    xv = x_ref[pl.ds(c, 1), :]
    iota = lax.broadcasted_iota(jnp.int32, (VOCAB, ROWS), 0)
    val = (xv == iota).astype(jnp.int32)
    scratch[h] = val
    for k in range(K):
        _dma(scratch, out_hbm, sems, h, k, c).start()

    @pl.when(c == COLS - 1)
    def _drain():
        for k in range(K):
            _dma(scratch, out_hbm, sems, 1 - h, k, c - 1).wait()
            _dma(scratch, out_hbm, sems, h, k, c).wait()


def kernel(x):
    xt = x.astype(jnp.int32).T
    out_t = pl.pallas_call(
        _body,
        grid=(COLS,),
        in_specs=[pl.BlockSpec((COLS, ROWS), lambda c: (0, 0))],
        out_specs=pl.BlockSpec(memory_space=pl.ANY),
        out_shape=jax.ShapeDtypeStruct((COLS, VOCAB, ROWS), jnp.int32),
        scratch_shapes=[
            pltpu.VMEM((NBUF, VOCAB, ROWS), jnp.int32),
            pltpu.SemaphoreType.DMA((NBUF, K)),
        ],
    )(xt)
    return jnp.transpose(out_t, (2, 0, 1))


# speedup vs baseline: 1.0324x; 1.0324x over previous
import jax
import jax.numpy as jnp
from jax import lax
from jax.experimental import pallas as pl


ROWS = 4096
COLS = 20
VOCAB = 1000
VB = 200


def _onehot_block(x_ref, out_ref):
    c = pl.program_id(0)
    v0 = pl.program_id(1) * VB
    xv = x_ref[pl.ds(c, 1), :][:, None, :]
    iota = v0 + lax.broadcasted_iota(jnp.int32, (1, VB, ROWS), 1)
    out_ref[...] = (xv == iota).astype(jnp.int32)


def kernel(x):
    xt = x.astype(jnp.int32).T
    out_t = pl.pallas_call(
        _onehot_block,
        grid=(COLS, VOCAB // VB),
        in_specs=[pl.BlockSpec((COLS, ROWS), lambda c, v: (0, 0))],
        out_specs=pl.BlockSpec((1, VB, ROWS), lambda c, v: (c, v, 0)),
        out_shape=jax.ShapeDtypeStruct((COLS, VOCAB, ROWS), jnp.int32),
    )(xt)
    return jnp.transpose(out_t, (2, 0, 1))
